# baseline (speedup 1.0000x reference)
import functools

import jax
import jax.numpy as jnp
from jax.experimental import pallas as pl
from jax.experimental.pallas import tpu as pltpu


def _conv4_kernel(*refs, relu, has_l):
    if has_l:
        x_ref, xl_ref, w_ref, b_ref, o_ref, pad_ref = refs
    else:
        x_ref, w_ref, b_ref, o_ref, pad_ref = refs
        xl_ref = None
    H = x_ref.shape[1]
    W = x_ref.shape[2]
    C = x_ref.shape[3]
    OH, OW = H // 2, W // 2
    O = o_ref.shape[3]

    pad_ref[...] = jnp.zeros_like(pad_ref)
    pad_ref[1:H + 1, 1:W + 1, 0:C] = x_ref[0]
    if xl_ref is not None:
        pad_ref[1:H + 1, 1:W + 1, C:] = jnp.transpose(xl_ref[0], (1, 2, 0))

    Ct = pad_ref.shape[2]
    v = pad_ref[...].reshape(OH + 1, 2, OW + 1, 2, Ct)
    acc = None
    for ky in range(4):
        dy, py = ky // 2, ky % 2
        xrow = v[dy:dy + OH, py]
        for kx in range(4):
            dx, px = kx // 2, kx % 2
            xs = xrow[:, dx:dx + OW, px]
            p = jnp.dot(xs.reshape(OH * OW, Ct), w_ref[ky, kx],
                        preferred_element_type=jnp.float32)
            acc = p if acc is None else acc + p
    y = acc.reshape(OH, OW, O) + b_ref[0]
    if relu:
        y = jnp.maximum(y, 0.0)
    o_ref[0] = y.astype(o_ref.dtype)


def _conv4(x, w_oihw, b, *, relu, xl=None):
    B, H, W, C = x.shape
    CL = 0 if xl is None else xl.shape[1]
    Ct = C + CL
    O = w_oihw.shape[0]
    wt = jnp.transpose(w_oihw, (2, 3, 1, 0)).astype(jnp.bfloat16)
    body = functools.partial(_conv4_kernel, relu=relu, has_l=xl is not None)
    in_specs = [pl.BlockSpec((1, H, W, C), lambda i: (i, 0, 0, 0))]
    args = [x]
    if xl is not None:
        in_specs.append(pl.BlockSpec((1, CL, H, W), lambda i: (i, 0, 0, 0)))
        args.append(xl)
    in_specs += [
        pl.BlockSpec((4, 4, Ct, O), lambda i: (0, 0, 0, 0)),
        pl.BlockSpec((1, O), lambda i: (0, 0)),
    ]
    args += [wt, b.reshape(1, O).astype(jnp.float32)]
    return pl.pallas_call(
        body,
        out_shape=jax.ShapeDtypeStruct((B, H // 2, W // 2, O), jnp.bfloat16),
        grid=(B,),
        in_specs=in_specs,
        out_specs=pl.BlockSpec((1, H // 2, W // 2, O), lambda i: (i, 0, 0, 0)),
        scratch_shapes=[pltpu.VMEM((H + 2, W + 2, Ct), jnp.bfloat16)],
        compiler_params=pltpu.CompilerParams(
            dimension_semantics=("parallel",)),
        cost_estimate=pl.CostEstimate(
            flops=2 * B * (H // 2) * (W // 2) * 16 * Ct * O, transcendentals=0,
            bytes_accessed=2 * B * (H * W * Ct + (H // 2) * (W // 2) * O)),
    )(*args)


def _convl_kernel(z_ref, w_ref, b_ref, o_ref, *, hw2, nco):
    B = z_ref.shape[0]
    j = pl.program_id(0)
    b = b_ref[pl.ds(j * nco, nco), :]
    y = jnp.dot(z_ref[...], w_ref[...], preferred_element_type=jnp.float32)
    y = y.reshape(B, nco, hw2) + b.reshape(1, nco, 1)
    y = jnp.maximum(y, 0.0)
    o_ref[...] = y.reshape(B, nco * hw2).astype(o_ref.dtype)


def _convl_gemm(z, w_flat, b, *, nco, hw2):
    B, K = z.shape
    N = w_flat.shape[1]
    tn = nco * hw2
    body = functools.partial(_convl_kernel, hw2=hw2, nco=nco)
    return pl.pallas_call(
        body,
        out_shape=jax.ShapeDtypeStruct((B, N), jnp.bfloat16),
        grid=(N // tn,),
        in_specs=[
            pl.BlockSpec((B, K), lambda j: (0, 0)),
            pl.BlockSpec((K, tn), lambda j: (0, j)),
            pl.BlockSpec((K, 1), lambda j: (0, 0)),
        ],
        out_specs=pl.BlockSpec((B, tn), lambda j: (0, j)),
        compiler_params=pltpu.CompilerParams(
            dimension_semantics=("parallel",)),
        cost_estimate=pl.CostEstimate(
            flops=2 * B * K * N, transcendentals=0,
            bytes_accessed=4 * (B * K + K * N) + 2 * B * N),
    )(z, w_flat, b.reshape(-1, 1))


def _fc_heads_kernel(x_ref, wfc_ref, bfc_ref, wh1_ref, bh1_ref,
                     wh2_ref, bh2_ref, o1_ref, o2_ref, acc_ref):
    k = pl.program_id(0)

    @pl.when(k == 0)
    def _():
        acc_ref[...] = jnp.zeros_like(acc_ref)

    dn = (((1,), (1,)), ((), ()))
    acc_ref[...] += jax.lax.dot_general(
        x_ref[...], wfc_ref[...].astype(jnp.bfloat16), dn,
        preferred_element_type=jnp.float32)

    @pl.when(k == pl.num_programs(0) - 1)
    def _():
        body = jnp.maximum(acc_ref[...] + bfc_ref[...], 0.0)
        l1 = jnp.dot(body, wh1_ref[...],
                     preferred_element_type=jnp.float32)
        o1_ref[...] = 1.0 / (1.0 + jnp.exp(-(l1 + bh1_ref[...])))
        l2 = jnp.dot(body, wh2_ref[...],
                     preferred_element_type=jnp.float32)
        l2 = l2 + bh2_ref[...]
        l2 = l2 - jnp.max(l2, axis=-1, keepdims=True)
        e2 = jnp.exp(l2)
        o2_ref[...] = e2 / jnp.sum(e2, axis=-1, keepdims=True)


def _fc_heads(x, wfc, bfc, wh1, bh1, wh2, bh2, *, tk):
    B, K = x.shape
    H = wfc.shape[0]
    wh1 = wh1.T
    wh2 = wh2.T
    n1, n2 = wh1.shape[1], wh2.shape[1]
    return pl.pallas_call(
        _fc_heads_kernel,
        out_shape=(jax.ShapeDtypeStruct((B, n1), jnp.float32),
                   jax.ShapeDtypeStruct((B, n2), jnp.float32)),
        grid=(K // tk,),
        in_specs=[
            pl.BlockSpec((B, tk), lambda k: (0, k)),
            pl.BlockSpec((H, tk), lambda k: (0, k)),
            pl.BlockSpec((1, H), lambda k: (0, 0)),
            pl.BlockSpec((H, n1), lambda k: (0, 0)),
            pl.BlockSpec((1, n1), lambda k: (0, 0)),
            pl.BlockSpec((H, n2), lambda k: (0, 0)),
            pl.BlockSpec((1, n2), lambda k: (0, 0)),
        ],
        out_specs=(pl.BlockSpec((B, n1), lambda k: (0, 0)),
                   pl.BlockSpec((B, n2), lambda k: (0, 0))),
        scratch_shapes=[pltpu.VMEM((B, H), jnp.float32)],
        compiler_params=pltpu.CompilerParams(
            dimension_semantics=("arbitrary",)),
        cost_estimate=pl.CostEstimate(
            flops=2 * B * K * H + 2 * B * H * (n1 + n2),
            transcendentals=B * (n1 + n2),
            bytes_accessed=4 * (K * H + H * (n1 + n2)) + 2 * B * K),
    )(x, wfc, bfc.reshape(1, H), wh1, bh1.reshape(1, n1),
      wh2, bh2.reshape(1, n2))


def kernel(w_conv_img, b_conv_img, w_conv_l, b_conv_l, w_conv2, b_conv2,
           w_conv3, b_conv3, w_conv4, b_conv4, w_fc, b_fc, w_h1, b_h1,
           w_h2, b_h2, img, age, gender):
    B = img.shape[0]
    n_l = age.shape[1]
    n_zch = w_conv_l.shape[0]
    n_age = (n_zch - 25) // n_l if n_l else 5
    n_gender = n_zch - n_l * n_age
    HW = w_conv_l.shape[2]

    l = jnp.tile(age, (1, n_age, 1, 1))
    g = jnp.tile(gender, (1, n_gender, 1, 1))
    z = jnp.concatenate([l, g], axis=1).reshape(B, n_zch)

    xi = jnp.transpose(img, (0, 2, 3, 1)).astype(jnp.bfloat16)
    y1 = _conv4(xi, w_conv_img, b_conv_img, relu=False)

    w_flat = w_conv_l.reshape(n_zch, n_zch * HW * HW)
    yl = _convl_gemm(z, w_flat, b_conv_l, nco=3, hw2=HW * HW)
    yl = yl.reshape(B, n_zch, HW, HW)

    h = _conv4(y1, w_conv2, b_conv2, relu=True, xl=yl)
    h = _conv4(h, w_conv3, b_conv3, relu=True)
    h = _conv4(h, w_conv4, b_conv4, relu=True)

    flat = jnp.transpose(h, (0, 3, 1, 2)).reshape(B, -1)
    return _fc_heads(flat, w_fc, b_fc, w_h1, b_h1, w_h2, b_h2, tk=2048)


# speedup vs baseline: 1.5488x; 1.5488x over previous
import functools

import jax
import jax.numpy as jnp
from jax.experimental import pallas as pl
from jax.experimental.pallas import tpu as pltpu


def _conv4_s2d_kernel(x_ref, w_ref, b_ref, o_ref, *, relu):
    oh = o_ref.shape[1]
    ow = o_ref.shape[2]
    c4 = x_ref.shape[3]
    acc = None
    t = 0
    for dy in range(2):
        for dx in range(2):
            xs = x_ref[0, dy:dy + oh, dx:dx + ow, :]
            xs2 = xs.reshape(oh * ow, c4)
            w = w_ref[t]
            p = jnp.dot(xs2, w, preferred_element_type=jnp.float32)
            acc = p if acc is None else acc + p
            t += 1
    y = acc.reshape(oh, ow, o_ref.shape[3]) + b_ref[0]
    if relu:
        y = jnp.maximum(y, 0.0)
    o_ref[0] = y.astype(o_ref.dtype)


def _conv4_s2d(x_s2d, w_taps, b, *, relu):
    B, S, _, C4 = x_s2d.shape
    O = w_taps.shape[2]
    OH = S - 1
    body = functools.partial(_conv4_s2d_kernel, relu=relu)
    return pl.pallas_call(
        body,
        out_shape=jax.ShapeDtypeStruct((B, OH, OH, O), jnp.bfloat16),
        grid=(B,),
        in_specs=[
            pl.BlockSpec((1, S, S, C4), lambda i: (i, 0, 0, 0)),
            pl.BlockSpec((4, C4, O), lambda i: (0, 0, 0)),
            pl.BlockSpec((1, O), lambda i: (0, 0)),
        ],
        out_specs=pl.BlockSpec((1, OH, OH, O), lambda i: (i, 0, 0, 0)),
        compiler_params=pltpu.CompilerParams(
            dimension_semantics=("parallel",)),
        cost_estimate=pl.CostEstimate(
            flops=2 * B * OH * OH * 4 * C4 * O, transcendentals=0,
            bytes_accessed=2 * (B * S * S * C4 + B * OH * OH * O) + 2 * 4 * C4 * O),
    )(x_s2d, w_taps, b.reshape(1, O).astype(jnp.float32))


def _s2d_pad1(x_nhwc):
    B, H, W, C = x_nhwc.shape
    S = H // 2 + 1
    xp = jnp.pad(x_nhwc, ((0, 0), (1, 1), (1, 1), (0, 0)))
    xp = xp.reshape(B, S, 2, S, 2, C)
    xp = jnp.transpose(xp, (0, 1, 3, 2, 4, 5))
    return xp.reshape(B, S, S, 4 * C).astype(jnp.bfloat16)


def _tap_weights(w_oihw):
    O, I, _, _ = w_oihw.shape
    wt = jnp.transpose(w_oihw, (2, 3, 1, 0))
    taps = [wt[2 * dy:2 * dy + 2, 2 * dx:2 * dx + 2].reshape(4 * I, O)
            for dy in range(2) for dx in range(2)]
    return jnp.stack(taps).astype(jnp.bfloat16)


def _convl_kernel(z_ref, w_ref, b_ref, o_ref, *, hw2, nco):
    B = z_ref.shape[0]
    j = pl.program_id(0)
    b = b_ref[pl.ds(j * nco, nco), :]
    y = jnp.dot(z_ref[...], w_ref[...], preferred_element_type=jnp.float32)
    y = y.reshape(B, nco, hw2) + b.reshape(1, nco, 1)
    y = jnp.maximum(y, 0.0)
    o_ref[...] = y.reshape(B, nco * hw2).astype(o_ref.dtype)


def _convl_gemm(z, w_flat, b, *, nco, hw2):
    B, K = z.shape
    N = w_flat.shape[1]
    tn = nco * hw2
    body = functools.partial(_convl_kernel, hw2=hw2, nco=nco)
    return pl.pallas_call(
        body,
        out_shape=jax.ShapeDtypeStruct((B, N), jnp.bfloat16),
        grid=(N // tn,),
        in_specs=[
            pl.BlockSpec((B, K), lambda j: (0, 0)),
            pl.BlockSpec((K, tn), lambda j: (0, j)),
            pl.BlockSpec((K, 1), lambda j: (0, 0)),
        ],
        out_specs=pl.BlockSpec((B, tn), lambda j: (0, j)),
        compiler_params=pltpu.CompilerParams(
            dimension_semantics=("parallel",)),
        cost_estimate=pl.CostEstimate(
            flops=2 * B * K * N, transcendentals=0,
            bytes_accessed=4 * (B * K + K * N) + 2 * B * N),
    )(z, w_flat, b.reshape(-1, 1))


def _nchw_to_nhwc_kernel(x_ref, o_ref):
    o_ref[0] = jnp.transpose(x_ref[0], (1, 2, 0))


def _nchw_to_nhwc(x):
    B, C, H, W = x.shape
    return pl.pallas_call(
        _nchw_to_nhwc_kernel,
        out_shape=jax.ShapeDtypeStruct((B, H, W, C), x.dtype),
        grid=(B,),
        in_specs=[pl.BlockSpec((1, C, H, W), lambda i: (i, 0, 0, 0))],
        out_specs=pl.BlockSpec((1, H, W, C), lambda i: (i, 0, 0, 0)),
        compiler_params=pltpu.CompilerParams(
            dimension_semantics=("parallel",)),
    )(x)


def _fc_heads_kernel(x_ref, wfc_ref, bfc_ref, wh1_ref, bh1_ref,
                     wh2_ref, bh2_ref, o1_ref, o2_ref, acc_ref):
    k = pl.program_id(0)

    @pl.when(k == 0)
    def _():
        acc_ref[...] = jnp.zeros_like(acc_ref)

    dn = (((1,), (1,)), ((), ()))
    acc_ref[...] += jax.lax.dot_general(
        x_ref[...], wfc_ref[...].astype(jnp.bfloat16), dn,
        preferred_element_type=jnp.float32)

    @pl.when(k == pl.num_programs(0) - 1)
    def _():
        body = jnp.maximum(acc_ref[...] + bfc_ref[...], 0.0)
        l1 = jnp.dot(body, wh1_ref[...],
                     preferred_element_type=jnp.float32)
        o1_ref[...] = 1.0 / (1.0 + jnp.exp(-(l1 + bh1_ref[...])))
        l2 = jnp.dot(body, wh2_ref[...],
                     preferred_element_type=jnp.float32)
        l2 = l2 + bh2_ref[...]
        l2 = l2 - jnp.max(l2, axis=-1, keepdims=True)
        e2 = jnp.exp(l2)
        o2_ref[...] = e2 / jnp.sum(e2, axis=-1, keepdims=True)


def _fc_heads(x, wfc, bfc, wh1, bh1, wh2, bh2, *, tk):
    B, K = x.shape
    H = wfc.shape[0]
    wh1 = wh1.T
    wh2 = wh2.T
    n1, n2 = wh1.shape[1], wh2.shape[1]
    return pl.pallas_call(
        _fc_heads_kernel,
        out_shape=(jax.ShapeDtypeStruct((B, n1), jnp.float32),
                   jax.ShapeDtypeStruct((B, n2), jnp.float32)),
        grid=(K // tk,),
        in_specs=[
            pl.BlockSpec((B, tk), lambda k: (0, k)),
            pl.BlockSpec((H, tk), lambda k: (0, k)),
            pl.BlockSpec((1, H), lambda k: (0, 0)),
            pl.BlockSpec((H, n1), lambda k: (0, 0)),
            pl.BlockSpec((1, n1), lambda k: (0, 0)),
            pl.BlockSpec((H, n2), lambda k: (0, 0)),
            pl.BlockSpec((1, n2), lambda k: (0, 0)),
        ],
        out_specs=(pl.BlockSpec((B, n1), lambda k: (0, 0)),
                   pl.BlockSpec((B, n2), lambda k: (0, 0))),
        scratch_shapes=[pltpu.VMEM((B, H), jnp.float32)],
        compiler_params=pltpu.CompilerParams(
            dimension_semantics=("arbitrary",)),
        cost_estimate=pl.CostEstimate(
            flops=2 * B * K * H + 2 * B * H * (n1 + n2),
            transcendentals=B * (n1 + n2),
            bytes_accessed=4 * (K * H + H * (n1 + n2)) + 2 * B * K),
    )(x, wfc, bfc.reshape(1, H), wh1, bh1.reshape(1, n1),
      wh2, bh2.reshape(1, n2))


def kernel(w_conv_img, b_conv_img, w_conv_l, b_conv_l, w_conv2, b_conv2,
           w_conv3, b_conv3, w_conv4, b_conv4, w_fc, b_fc, w_h1, b_h1,
           w_h2, b_h2, img, age, gender):
    B = img.shape[0]
    n_l = age.shape[1]
    n_zch = w_conv_l.shape[0]
    n_age = (n_zch - 25) // n_l if n_l else 5
    n_gender = n_zch - n_l * n_age
    HW = w_conv_l.shape[2]

    l = jnp.tile(age, (1, n_age, 1, 1))
    g = jnp.tile(gender, (1, n_gender, 1, 1))
    z = jnp.concatenate([l, g], axis=1).reshape(B, n_zch)

    xs = _s2d_pad1(jnp.transpose(img, (0, 2, 3, 1)))
    y1 = _conv4_s2d(xs, _tap_weights(w_conv_img), b_conv_img, relu=False)

    w_flat = w_conv_l.reshape(n_zch, n_zch * HW * HW)
    yl = _convl_gemm(z, w_flat, b_conv_l, nco=3, hw2=HW * HW)
    yl = _nchw_to_nhwc(yl.reshape(B, n_zch, HW, HW))

    h = jnp.concatenate([y1, yl], axis=-1)
    h = _conv4_s2d(_s2d_pad1(h), _tap_weights(w_conv2), b_conv2, relu=True)
    h = _conv4_s2d(_s2d_pad1(h), _tap_weights(w_conv3), b_conv3, relu=True)
    h = _conv4_s2d(_s2d_pad1(h), _tap_weights(w_conv4), b_conv4, relu=True)

    flat = jnp.transpose(h, (0, 3, 1, 2)).reshape(B, -1)
    return _fc_heads(flat, w_fc, b_fc, w_h1, b_h1, w_h2, b_h2, tk=2048)


# speedup vs baseline: 1.6444x; 1.0617x over previous
import functools

import jax
import jax.numpy as jnp
from jax.experimental import pallas as pl
from jax.experimental.pallas import tpu as pltpu


def _conv4_s2d_kernel(x_ref, w_ref, b_ref, o_ref, *, relu):
    bb = o_ref.shape[0]
    oh = o_ref.shape[1]
    ow = o_ref.shape[2]
    c4 = x_ref.shape[3]
    acc = None
    t = 0
    for dy in range(2):
        for dx in range(2):
            xs = x_ref[:, dy:dy + oh, dx:dx + ow, :]
            xs2 = xs.reshape(bb * oh * ow, c4)
            w = w_ref[t]
            p = jnp.dot(xs2, w, preferred_element_type=jnp.float32)
            acc = p if acc is None else acc + p
            t += 1
    y = acc.reshape(bb, oh, ow, o_ref.shape[3]) + b_ref[0]
    if relu:
        y = jnp.maximum(y, 0.0)
    o_ref[...] = y.astype(o_ref.dtype)


def _conv4_s2d(x_s2d, w_taps, b, *, relu, bb=4):
    B, S, _, C4 = x_s2d.shape
    O = w_taps.shape[2]
    OH = S - 1
    body = functools.partial(_conv4_s2d_kernel, relu=relu)
    return pl.pallas_call(
        body,
        out_shape=jax.ShapeDtypeStruct((B, OH, OH, O), jnp.bfloat16),
        grid=(B // bb,),
        in_specs=[
            pl.BlockSpec((bb, S, S, C4), lambda i: (i, 0, 0, 0)),
            pl.BlockSpec((4, C4, O), lambda i: (0, 0, 0)),
            pl.BlockSpec((1, O), lambda i: (0, 0)),
        ],
        out_specs=pl.BlockSpec((bb, OH, OH, O), lambda i: (i, 0, 0, 0)),
        compiler_params=pltpu.CompilerParams(
            dimension_semantics=("parallel",)),
        cost_estimate=pl.CostEstimate(
            flops=2 * B * OH * OH * 4 * C4 * O, transcendentals=0,
            bytes_accessed=2 * (B * S * S * C4 + B * OH * OH * O) + 2 * 4 * C4 * O),
    )(x_s2d, w_taps, b.reshape(1, O).astype(jnp.float32))


def _s2d_pad1(x_nhwc):
    B, H, W, C = x_nhwc.shape
    S = H // 2 + 1
    xp = jnp.pad(x_nhwc, ((0, 0), (1, 1), (1, 1), (0, 0)))
    xp = xp.reshape(B, S, 2, S, 2, C)
    xp = jnp.transpose(xp, (0, 1, 3, 2, 4, 5))
    return xp.reshape(B, S, S, 4 * C).astype(jnp.bfloat16)


def _tap_weights(w_oihw):
    O, I, _, _ = w_oihw.shape
    wt = jnp.transpose(w_oihw, (2, 3, 1, 0))
    taps = [wt[2 * dy:2 * dy + 2, 2 * dx:2 * dx + 2].reshape(4 * I, O)
            for dy in range(2) for dx in range(2)]
    return jnp.stack(taps).astype(jnp.bfloat16)


def _convl_kernel(z_ref, w_ref, b_ref, o_ref, *, hw2, nco):
    B = z_ref.shape[0]
    j = pl.program_id(0)
    b = b_ref[pl.ds(j * nco, nco), :]
    y = jnp.dot(z_ref[...], w_ref[...], preferred_element_type=jnp.float32)
    y = y.reshape(B, nco, hw2) + b.reshape(1, nco, 1)
    y = jnp.maximum(y, 0.0)
    o_ref[...] = y.reshape(B, nco * hw2).astype(o_ref.dtype)


def _convl_gemm(z, w_flat, b, *, nco, hw2):
    B, K = z.shape
    N = w_flat.shape[1]
    tn = nco * hw2
    body = functools.partial(_convl_kernel, hw2=hw2, nco=nco)
    return pl.pallas_call(
        body,
        out_shape=jax.ShapeDtypeStruct((B, N), jnp.bfloat16),
        grid=(N // tn,),
        in_specs=[
            pl.BlockSpec((B, K), lambda j: (0, 0)),
            pl.BlockSpec((K, tn), lambda j: (0, j)),
            pl.BlockSpec((K, 1), lambda j: (0, 0)),
        ],
        out_specs=pl.BlockSpec((B, tn), lambda j: (0, j)),
        compiler_params=pltpu.CompilerParams(
            dimension_semantics=("parallel",)),
        cost_estimate=pl.CostEstimate(
            flops=2 * B * K * N, transcendentals=0,
            bytes_accessed=4 * (B * K + K * N) + 2 * B * N),
    )(z, w_flat, b.reshape(-1, 1))


def _nchw_to_nhwc_kernel(x_ref, o_ref):
    o_ref[...] = jnp.transpose(x_ref[...], (0, 2, 3, 1))


def _nchw_to_nhwc(x, *, bb=4):
    B, C, H, W = x.shape
    return pl.pallas_call(
        _nchw_to_nhwc_kernel,
        out_shape=jax.ShapeDtypeStruct((B, H, W, C), x.dtype),
        grid=(B // bb,),
        in_specs=[pl.BlockSpec((bb, C, H, W), lambda i: (i, 0, 0, 0))],
        out_specs=pl.BlockSpec((bb, H, W, C), lambda i: (i, 0, 0, 0)),
        compiler_params=pltpu.CompilerParams(
            dimension_semantics=("parallel",)),
    )(x)


def _fc_heads_kernel(x_ref, wfc_ref, bfc_ref, wh1_ref, bh1_ref,
                     wh2_ref, bh2_ref, o1_ref, o2_ref, acc_ref):
    k = pl.program_id(0)

    @pl.when(k == 0)
    def _():
        acc_ref[...] = jnp.zeros_like(acc_ref)

    dn = (((1,), (1,)), ((), ()))
    acc_ref[...] += jax.lax.dot_general(
        x_ref[...], wfc_ref[...].astype(jnp.bfloat16), dn,
        preferred_element_type=jnp.float32)

    @pl.when(k == pl.num_programs(0) - 1)
    def _():
        body = jnp.maximum(acc_ref[...] + bfc_ref[...], 0.0)
        l1 = jnp.dot(body, wh1_ref[...],
                     preferred_element_type=jnp.float32)
        o1_ref[...] = 1.0 / (1.0 + jnp.exp(-(l1 + bh1_ref[...])))
        l2 = jnp.dot(body, wh2_ref[...],
                     preferred_element_type=jnp.float32)
        l2 = l2 + bh2_ref[...]
        l2 = l2 - jnp.max(l2, axis=-1, keepdims=True)
        e2 = jnp.exp(l2)
        o2_ref[...] = e2 / jnp.sum(e2, axis=-1, keepdims=True)


def _fc_heads(x, wfc, bfc, wh1, bh1, wh2, bh2, *, tk):
    B, K = x.shape
    H = wfc.shape[0]
    wh1 = wh1.T
    wh2 = wh2.T
    n1, n2 = wh1.shape[1], wh2.shape[1]
    return pl.pallas_call(
        _fc_heads_kernel,
        out_shape=(jax.ShapeDtypeStruct((B, n1), jnp.float32),
                   jax.ShapeDtypeStruct((B, n2), jnp.float32)),
        grid=(K // tk,),
        in_specs=[
            pl.BlockSpec((B, tk), lambda k: (0, k)),
            pl.BlockSpec((H, tk), lambda k: (0, k)),
            pl.BlockSpec((1, H), lambda k: (0, 0)),
            pl.BlockSpec((H, n1), lambda k: (0, 0)),
            pl.BlockSpec((1, n1), lambda k: (0, 0)),
            pl.BlockSpec((H, n2), lambda k: (0, 0)),
            pl.BlockSpec((1, n2), lambda k: (0, 0)),
        ],
        out_specs=(pl.BlockSpec((B, n1), lambda k: (0, 0)),
                   pl.BlockSpec((B, n2), lambda k: (0, 0))),
        scratch_shapes=[pltpu.VMEM((B, H), jnp.float32)],
        compiler_params=pltpu.CompilerParams(
            dimension_semantics=("arbitrary",)),
        cost_estimate=pl.CostEstimate(
            flops=2 * B * K * H + 2 * B * H * (n1 + n2),
            transcendentals=B * (n1 + n2),
            bytes_accessed=4 * (K * H + H * (n1 + n2)) + 2 * B * K),
    )(x, wfc, bfc.reshape(1, H), wh1, bh1.reshape(1, n1),
      wh2, bh2.reshape(1, n2))


def kernel(w_conv_img, b_conv_img, w_conv_l, b_conv_l, w_conv2, b_conv2,
           w_conv3, b_conv3, w_conv4, b_conv4, w_fc, b_fc, w_h1, b_h1,
           w_h2, b_h2, img, age, gender):
    B = img.shape[0]
    n_l = age.shape[1]
    n_zch = w_conv_l.shape[0]
    n_age = (n_zch - 25) // n_l if n_l else 5
    n_gender = n_zch - n_l * n_age
    HW = w_conv_l.shape[2]

    l = jnp.tile(age, (1, n_age, 1, 1))
    g = jnp.tile(gender, (1, n_gender, 1, 1))
    z = jnp.concatenate([l, g], axis=1).reshape(B, n_zch)

    xs = _s2d_pad1(jnp.transpose(img, (0, 2, 3, 1)))
    y1 = _conv4_s2d(xs, _tap_weights(w_conv_img), b_conv_img, relu=False)

    w_flat = w_conv_l.reshape(n_zch, n_zch * HW * HW)
    yl = _convl_gemm(z, w_flat, b_conv_l, nco=3, hw2=HW * HW)
    yl = _nchw_to_nhwc(yl.reshape(B, n_zch, HW, HW))

    h = jnp.concatenate([y1, yl], axis=-1)
    h = _conv4_s2d(_s2d_pad1(h), _tap_weights(w_conv2), b_conv2, relu=True)
    h = _conv4_s2d(_s2d_pad1(h), _tap_weights(w_conv3), b_conv3, relu=True)
    h = _conv4_s2d(_s2d_pad1(h), _tap_weights(w_conv4), b_conv4, relu=True)

    flat = jnp.transpose(h, (0, 3, 1, 2)).reshape(B, -1)
    return _fc_heads(flat, w_fc, b_fc, w_h1, b_h1, w_h2, b_h2, tk=2048)


# speedup vs baseline: 1.6459x; 1.0009x over previous
import functools

import jax
import jax.numpy as jnp
from jax.experimental import pallas as pl
from jax.experimental.pallas import tpu as pltpu


def _conv4_s2d_kernel(x_ref, w_ref, b_ref, o_ref, *, relu):
    bb = o_ref.shape[0]
    oh = o_ref.shape[1]
    ow = o_ref.shape[2]
    c4 = x_ref.shape[3]
    acc = None
    t = 0
    for dy in range(2):
        for dx in range(2):
            xs = x_ref[:, dy:dy + oh, dx:dx + ow, :]
            xs2 = xs.reshape(bb * oh * ow, c4)
            w = w_ref[t]
            p = jnp.dot(xs2, w, preferred_element_type=jnp.float32)
            acc = p if acc is None else acc + p
            t += 1
    y = acc.reshape(bb, oh, ow, o_ref.shape[3]) + b_ref[0]
    if relu:
        y = jnp.maximum(y, 0.0)
    o_ref[...] = y.astype(o_ref.dtype)


def _conv4_s2d(x_s2d, w_taps, b, *, relu, bb=4):
    B, S, _, C4 = x_s2d.shape
    O = w_taps.shape[2]
    OH = S - 1
    body = functools.partial(_conv4_s2d_kernel, relu=relu)
    return pl.pallas_call(
        body,
        out_shape=jax.ShapeDtypeStruct((B, OH, OH, O), jnp.bfloat16),
        grid=(B // bb,),
        in_specs=[
            pl.BlockSpec((bb, S, S, C4), lambda i: (i, 0, 0, 0)),
            pl.BlockSpec((4, C4, O), lambda i: (0, 0, 0)),
            pl.BlockSpec((1, O), lambda i: (0, 0)),
        ],
        out_specs=pl.BlockSpec((bb, OH, OH, O), lambda i: (i, 0, 0, 0)),
        compiler_params=pltpu.CompilerParams(
            dimension_semantics=("parallel",)),
        cost_estimate=pl.CostEstimate(
            flops=2 * B * OH * OH * 4 * C4 * O, transcendentals=0,
            bytes_accessed=2 * (B * S * S * C4 + B * OH * OH * O) + 2 * 4 * C4 * O),
    )(x_s2d, w_taps, b.reshape(1, O).astype(jnp.float32))


def _s2d_pad1(x_nhwc):
    B, H, W, C = x_nhwc.shape
    S = H // 2 + 1
    xp = jnp.pad(x_nhwc, ((0, 0), (1, 1), (1, 1), (0, 0)))
    xp = xp.reshape(B, S, 2, S, 2, C)
    xp = jnp.transpose(xp, (0, 1, 3, 2, 4, 5))
    return xp.reshape(B, S, S, 4 * C).astype(jnp.bfloat16)


def _tap_weights(w_oihw):
    O, I, _, _ = w_oihw.shape
    wt = jnp.transpose(w_oihw, (2, 3, 1, 0))
    taps = [wt[2 * dy:2 * dy + 2, 2 * dx:2 * dx + 2].reshape(4 * I, O)
            for dy in range(2) for dx in range(2)]
    return jnp.stack(taps).astype(jnp.bfloat16)


def _convl_kernel(z_ref, w_ref, b_ref, o_ref, *, hw2, nco):
    B = z_ref.shape[0]
    j = pl.program_id(0)
    b = b_ref[pl.ds(j * nco, nco), :]
    y = jnp.dot(z_ref[...], w_ref[...], preferred_element_type=jnp.float32)
    y = y.reshape(B, nco, hw2) + b.reshape(1, nco, 1)
    y = jnp.maximum(y, 0.0)
    o_ref[...] = y.reshape(B, nco * hw2).astype(o_ref.dtype)


def _convl_gemm(z, w_flat, b, *, nco, hw2):
    B, K = z.shape
    N = w_flat.shape[1]
    tn = nco * hw2
    body = functools.partial(_convl_kernel, hw2=hw2, nco=nco)
    return pl.pallas_call(
        body,
        out_shape=jax.ShapeDtypeStruct((B, N), jnp.bfloat16),
        grid=(N // tn,),
        in_specs=[
            pl.BlockSpec((B, K), lambda j: (0, 0)),
            pl.BlockSpec((K, tn), lambda j: (0, j)),
            pl.BlockSpec((K, 1), lambda j: (0, 0)),
        ],
        out_specs=pl.BlockSpec((B, tn), lambda j: (0, j)),
        compiler_params=pltpu.CompilerParams(
            dimension_semantics=("parallel",)),
        cost_estimate=pl.CostEstimate(
            flops=2 * B * K * N, transcendentals=0,
            bytes_accessed=4 * (B * K + K * N) + 2 * B * N),
    )(z, w_flat, b.reshape(-1, 1))


def _nchw_to_nhwc_kernel(x_ref, o_ref):
    o_ref[...] = jnp.transpose(x_ref[...], (0, 2, 3, 1))


def _nchw_to_nhwc(x, *, bb=4):
    B, C, H, W = x.shape
    return pl.pallas_call(
        _nchw_to_nhwc_kernel,
        out_shape=jax.ShapeDtypeStruct((B, H, W, C), x.dtype),
        grid=(B // bb,),
        in_specs=[pl.BlockSpec((bb, C, H, W), lambda i: (i, 0, 0, 0))],
        out_specs=pl.BlockSpec((bb, H, W, C), lambda i: (i, 0, 0, 0)),
        compiler_params=pltpu.CompilerParams(
            dimension_semantics=("parallel",)),
    )(x)


def _fc_heads_kernel(x_ref, wfc_ref, bfc_ref, wh1_ref, bh1_ref,
                     wh2_ref, bh2_ref, o1_ref, o2_ref, acc_ref):
    k = pl.program_id(0)

    @pl.when(k == 0)
    def _():
        acc_ref[...] = jnp.zeros_like(acc_ref)

    dn = (((1,), (1,)), ((), ()))
    acc_ref[...] += jax.lax.dot_general(
        x_ref[...], wfc_ref[...].astype(jnp.bfloat16), dn,
        preferred_element_type=jnp.float32)

    @pl.when(k == pl.num_programs(0) - 1)
    def _():
        body = jnp.maximum(acc_ref[...] + bfc_ref[...], 0.0)
        l1 = jnp.dot(body, wh1_ref[...],
                     preferred_element_type=jnp.float32)
        o1_ref[...] = 1.0 / (1.0 + jnp.exp(-(l1 + bh1_ref[...])))
        l2 = jnp.dot(body, wh2_ref[...],
                     preferred_element_type=jnp.float32)
        l2 = l2 + bh2_ref[...]
        l2 = l2 - jnp.max(l2, axis=-1, keepdims=True)
        e2 = jnp.exp(l2)
        o2_ref[...] = e2 / jnp.sum(e2, axis=-1, keepdims=True)


def _fc_heads(x, wfc, bfc, wh1, bh1, wh2, bh2, *, tk):
    B, K = x.shape
    H = wfc.shape[0]
    wh1 = wh1.T
    wh2 = wh2.T
    n1, n2 = wh1.shape[1], wh2.shape[1]
    return pl.pallas_call(
        _fc_heads_kernel,
        out_shape=(jax.ShapeDtypeStruct((B, n1), jnp.float32),
                   jax.ShapeDtypeStruct((B, n2), jnp.float32)),
        grid=(K // tk,),
        in_specs=[
            pl.BlockSpec((B, tk), lambda k: (0, k)),
            pl.BlockSpec((H, tk), lambda k: (0, k)),
            pl.BlockSpec((1, H), lambda k: (0, 0)),
            pl.BlockSpec((H, n1), lambda k: (0, 0)),
            pl.BlockSpec((1, n1), lambda k: (0, 0)),
            pl.BlockSpec((H, n2), lambda k: (0, 0)),
            pl.BlockSpec((1, n2), lambda k: (0, 0)),
        ],
        out_specs=(pl.BlockSpec((B, n1), lambda k: (0, 0)),
                   pl.BlockSpec((B, n2), lambda k: (0, 0))),
        scratch_shapes=[pltpu.VMEM((B, H), jnp.float32)],
        compiler_params=pltpu.CompilerParams(
            dimension_semantics=("arbitrary",)),
        cost_estimate=pl.CostEstimate(
            flops=2 * B * K * H + 2 * B * H * (n1 + n2),
            transcendentals=B * (n1 + n2),
            bytes_accessed=4 * (K * H + H * (n1 + n2)) + 2 * B * K),
    )(x, wfc, bfc.reshape(1, H), wh1, bh1.reshape(1, n1),
      wh2, bh2.reshape(1, n2))


def kernel(w_conv_img, b_conv_img, w_conv_l, b_conv_l, w_conv2, b_conv2,
           w_conv3, b_conv3, w_conv4, b_conv4, w_fc, b_fc, w_h1, b_h1,
           w_h2, b_h2, img, age, gender):
    B = img.shape[0]
    n_l = age.shape[1]
    n_zch = w_conv_l.shape[0]
    n_age = (n_zch - 25) // n_l if n_l else 5
    n_gender = n_zch - n_l * n_age
    HW = w_conv_l.shape[2]

    l = jnp.tile(age, (1, n_age, 1, 1))
    g = jnp.tile(gender, (1, n_gender, 1, 1))
    z = jnp.concatenate([l, g], axis=1).reshape(B, n_zch)

    xs = _s2d_pad1(jnp.transpose(img, (0, 2, 3, 1)))
    y1 = _conv4_s2d(xs, _tap_weights(w_conv_img), b_conv_img, relu=False)

    w_flat = w_conv_l.reshape(n_zch, n_zch * HW * HW)
    yl = _convl_gemm(z, w_flat, b_conv_l, nco=5, hw2=HW * HW)
    yl = _nchw_to_nhwc(yl.reshape(B, n_zch, HW, HW))

    h = jnp.concatenate([y1, yl], axis=-1)
    h = _conv4_s2d(_s2d_pad1(h), _tap_weights(w_conv2), b_conv2, relu=True)
    h = _conv4_s2d(_s2d_pad1(h), _tap_weights(w_conv3), b_conv3, relu=True)
    h = _conv4_s2d(_s2d_pad1(h), _tap_weights(w_conv4), b_conv4, relu=True)

    flat = jnp.transpose(h, (0, 3, 1, 2)).reshape(B, -1)
    return _fc_heads(flat, w_fc, b_fc, w_h1, b_h1, w_h2, b_h2, tk=2048)
